# baseline (speedup 1.0000x reference)
import jax
import jax.numpy as jnp
from jax.experimental import pallas as pl
from jax.experimental.pallas import tpu as pltpu

_N = 5000
_K = 12544
_H = 1024
_BN = 1680
_BK = 1792
_NK = _K // _BK
_NN = 3

_DN = (((1,), (1,)), ((), ()))
_DT = (((1,), (0,)), ((), ()))


def _body(x_ref, w1_ref, b1_ref, w2_ref, b2_ref, wh_ref, bh_ref,
          cls_ref, box_ref, acc_ref):
    k = pl.program_id(1)

    part = jax.lax.dot_general(
        x_ref[...], w1_ref[...], _DN, preferred_element_type=jnp.float32)

    @pl.when(k == 0)
    def _init():
        acc_ref[...] = part

    @pl.when(k > 0)
    def _accum():
        acc_ref[...] += part

    @pl.when(k == _NK - 1)
    def _finish():
        third = _BN // 3
        for c in range(3):
            rows = pl.ds(c * third, third)
            h1 = jnp.maximum(acc_ref[rows, :] + b1_ref[...], 0.0)
            h2 = jax.lax.dot_general(
                h1, w2_ref[...], _DT, preferred_element_type=jnp.float32)
            h2 = jnp.maximum(h2 + b2_ref[...], 0.0)
            hh = jax.lax.dot_general(
                h2, wh_ref[...], _DT,
                preferred_element_type=jnp.float32) + bh_ref[...]
            cls_ref[rows, :] = hh[:, :4]
            box_ref[rows, :] = hh[:, 4:]


def kernel(feature_vectors, W1, b1, W2, b2, Wc, bc, Wr, br):
    c1 = Wc.shape[0]
    c4 = Wr.shape[0]
    ch = c1 + c4
    WhT = jnp.concatenate([Wc, Wr], axis=0).T.astype(jnp.bfloat16)
    bh = jnp.concatenate([bc, br], axis=0).reshape(1, -1)
    cls_out, box_out = pl.pallas_call(
        _body,
        grid=(_NN, _NK),
        in_specs=[
            pl.BlockSpec((_BN, _BK), lambda n, k: (n, k)),
            pl.BlockSpec((_H, _BK), lambda n, k: (0, k)),
            pl.BlockSpec((1, _H), lambda n, k: (0, 0)),
            pl.BlockSpec((_H, _H), lambda n, k: (0, 0)),
            pl.BlockSpec((1, _H), lambda n, k: (0, 0)),
            pl.BlockSpec((_H, ch), lambda n, k: (0, 0)),
            pl.BlockSpec((1, ch), lambda n, k: (0, 0)),
        ],
        out_specs=[
            pl.BlockSpec((_BN, c1), lambda n, k: (n, 0)),
            pl.BlockSpec((_BN, c4), lambda n, k: (n, 0)),
        ],
        out_shape=[
            jax.ShapeDtypeStruct((_N, c1), jnp.float32),
            jax.ShapeDtypeStruct((_N, c4), jnp.float32),
        ],
        scratch_shapes=[pltpu.VMEM((_BN, _H), jnp.float32)],
        compiler_params=pltpu.CompilerParams(
            dimension_semantics=("parallel", "arbitrary")),
    )(feature_vectors, W1, b1.reshape(1, -1), W2.T.astype(jnp.bfloat16),
      b2.reshape(1, -1), WhT, bh)
    return (cls_out, box_out)


# speedup vs baseline: 1.0074x; 1.0074x over previous
import jax
import jax.numpy as jnp
from jax.experimental import pallas as pl
from jax.experimental.pallas import tpu as pltpu

_N = 5000
_K = 12544
_H = 1024
_BN = 1680
_BK = 1792
_NK = _K // _BK
_NN = 3

_DN = (((1,), (1,)), ((), ()))


def _body(x_ref, w1_ref, b1_ref, w2_ref, b2_ref, wh_ref, bh_ref,
          cls_ref, box_ref, acc_ref):
    k = pl.program_id(1)

    part = jax.lax.dot_general(
        x_ref[...], w1_ref[...], _DN, preferred_element_type=jnp.float32)

    @pl.when(k == 0)
    def _init():
        acc_ref[...] = part

    @pl.when(k > 0)
    def _accum():
        acc_ref[...] += part

    @pl.when(k == _NK - 1)
    def _finish():
        third = _BN // 3
        for c in range(3):
            rows = pl.ds(c * third, third)
            h1 = jnp.maximum(acc_ref[rows, :] + b1_ref[...], 0.0)
            h2 = jax.lax.dot_general(
                h1, w2_ref[...], _DN, preferred_element_type=jnp.float32)
            h2 = jnp.maximum(h2 + b2_ref[...], 0.0)
            hh = jax.lax.dot_general(
                h2, wh_ref[...], _DN,
                preferred_element_type=jnp.float32) + bh_ref[...]
            cls_ref[rows, :] = hh[:, :4]
            box_ref[rows, :] = hh[:, 4:]


def kernel(feature_vectors, W1, b1, W2, b2, Wc, bc, Wr, br):
    c1 = Wc.shape[0]
    c4 = Wr.shape[0]
    ch = c1 + c4
    Wh = jnp.concatenate([Wc, Wr], axis=0).astype(jnp.bfloat16)
    bh = jnp.concatenate([bc, br], axis=0).reshape(1, -1)
    cls_out, box_out = pl.pallas_call(
        _body,
        grid=(_NN, _NK),
        in_specs=[
            pl.BlockSpec((_BN, _BK), lambda n, k: (n, k)),
            pl.BlockSpec((_H, _BK), lambda n, k: (0, k)),
            pl.BlockSpec((1, _H), lambda n, k: (0, 0)),
            pl.BlockSpec((_H, _H), lambda n, k: (0, 0)),
            pl.BlockSpec((1, _H), lambda n, k: (0, 0)),
            pl.BlockSpec((ch, _H), lambda n, k: (0, 0)),
            pl.BlockSpec((1, ch), lambda n, k: (0, 0)),
        ],
        out_specs=[
            pl.BlockSpec((_BN, c1), lambda n, k: (n, 0)),
            pl.BlockSpec((_BN, c4), lambda n, k: (n, 0)),
        ],
        out_shape=[
            jax.ShapeDtypeStruct((_N, c1), jnp.float32),
            jax.ShapeDtypeStruct((_N, c4), jnp.float32),
        ],
        scratch_shapes=[pltpu.VMEM((_BN, _H), jnp.float32)],
        compiler_params=pltpu.CompilerParams(
            dimension_semantics=("parallel", "arbitrary")),
    )(feature_vectors, W1, b1.reshape(1, -1), W2.astype(jnp.bfloat16),
      b2.reshape(1, -1), Wh, bh)
    return (cls_out, box_out)
